# baseline (speedup 1.0000x reference)
import functools

import jax
import jax.numpy as jnp
from jax import lax
from jax.experimental import pallas as pl
from jax.experimental.pallas import tpu as pltpu
from jax.experimental.pallas import tpu_sc as plsc

VOCAB = 1000
EMB_DIM = 16
BATCH = 128

_NC = 2
_NW = 16 * _NC
_ROWS_PER_W = BATCH // _NW
_GPW = 4 * _ROWS_PER_W


@functools.cache
def _sc_gather_fn():
    mesh = plsc.VectorSubcoreMesh(
        core_axis_name="c", subcore_axis_name="s", num_cores=_NC
    )

    @functools.partial(
        pl.kernel,
        mesh=mesh,
        out_type=jax.ShapeDtypeStruct((4 * BATCH, EMB_DIM), jnp.float32),
        scratch_types=[
            pltpu.VMEM((_GPW,), jnp.float32),
            pltpu.VMEM((_GPW,), jnp.int32),
            pltpu.VMEM((_GPW, EMB_DIM), jnp.float32),
            pltpu.SemaphoreType.DMA,
        ],
        compiler_params=pltpu.CompilerParams(use_tc_tiling_on_sc=False),
    )
    def _sc_gather(table_hbm, x_hbm, out_hbm, xv, idx_v, rows_v, sem):
        s = lax.axis_index("s") * _NC + lax.axis_index("c")
        base = s * _GPW
        pltpu.sync_copy(x_hbm.at[pl.ds(base, _GPW)], xv)
        for half in range(_GPW // 16):
            v = xv[pl.ds(16 * half, 16)].astype(jnp.int32)
            idx_v[pl.ds(16 * half, 16)] = jnp.minimum(
                jnp.maximum(v, 0), VOCAB - 1
            )
        pltpu.async_copy(table_hbm.at[idx_v], rows_v, sem).wait()
        pltpu.sync_copy(rows_v, out_hbm.at[pl.ds(base, _GPW), :])

    return _sc_gather


def _dot_t(a, w):
    return lax.dot_general(
        a, w, (((1,), (1,)), ((), ())), preferred_element_type=jnp.float32
    )


def _mlp_body(e_ref, x_ref, w1_ref, b1_ref, w2_ref, b2_ref, w3_ref, b3_ref,
              w4_ref, b4_ref, out_ref):
    e = e_ref[...]
    w1 = w1_ref[...]
    w2, w3, w4 = w2_ref[...], w3_ref[...], w4_ref[...]
    b1, b2, b3 = b1_ref[...], b2_ref[...], b3_ref[...]
    x = x_ref[...]
    q = lax.broadcasted_iota(jnp.int32, (BATCH // 2, BATCH), 0)
    b = lax.broadcasted_iota(jnp.int32, (BATCH // 2, BATCH), 1)
    outs = []
    for p in range(2):
        emb = e[:, 64 * p : 64 * p + 2 * EMB_DIM]
        sel = (b == 2 * q + p).astype(jnp.float32)
        wr = sel @ x[:, 2:4]
        h = _dot_t(emb, w1[:, : 2 * EMB_DIM])
        h = h + _dot_t(wr, w1[:, 2 * EMB_DIM :]) + b1[None, :]
        h = jnp.maximum(h, 0.0)
        h = jnp.maximum(_dot_t(h, w2) + b2[None, :], 0.0)
        h = jnp.maximum(_dot_t(h, w3) + b3[None, :], 0.0)
        outs.append(jnp.sum(h * w4, axis=1, keepdims=True) + b4_ref[0])
    bb = lax.broadcasted_iota(jnp.int32, (BATCH, BATCH // 2), 0)
    qq = lax.broadcasted_iota(jnp.int32, (BATCH, BATCH // 2), 1)
    acc = jnp.zeros((BATCH, 1), jnp.float32)
    for p in range(2):
        selt = (bb == 2 * qq + p).astype(jnp.float32)
        acc = acc + lax.dot_general(
            selt, outs[p], (((1,), (0,)), ((), ())),
            preferred_element_type=jnp.float32,
        )
    out_ref[...] = acc


def kernel(x, emb_table, W1, b1, W2, b2, W3, b3, W4, b4):
    rows = _sc_gather_fn()(emb_table, x.reshape(4 * BATCH))
    e = rows.reshape(BATCH // 2, 4 * 2 * EMB_DIM)
    out = pl.pallas_call(
        _mlp_body,
        out_shape=jax.ShapeDtypeStruct((BATCH, 1), jnp.float32),
    )(e, x, W1, b1, W2, b2, W3, b3, W4, b4)
    return out


# speedup vs baseline: 1.0771x; 1.0771x over previous
import functools

import jax
import jax.numpy as jnp
from jax import lax
from jax.experimental import pallas as pl
from jax.experimental.pallas import tpu as pltpu
from jax.experimental.pallas import tpu_sc as plsc

VOCAB = 1000
EMB_DIM = 16
BATCH = 128

_NW = 16
_ROWS_PER_W = BATCH // _NW
_GPW = 4 * _ROWS_PER_W


@functools.cache
def _sc_gather_fn():
    mesh = plsc.VectorSubcoreMesh(
        core_axis_name="c", subcore_axis_name="s", num_cores=1
    )

    @functools.partial(
        pl.kernel,
        mesh=mesh,
        out_type=jax.ShapeDtypeStruct((4 * BATCH, EMB_DIM), jnp.float32),
        scratch_types=[
            pltpu.VMEM((_GPW,), jnp.float32),
            pltpu.VMEM((_GPW,), jnp.int32),
            pltpu.VMEM((_GPW, EMB_DIM), jnp.float32),
            pltpu.SemaphoreType.DMA,
        ],
        compiler_params=pltpu.CompilerParams(use_tc_tiling_on_sc=False),
    )
    def _sc_gather(table_hbm, x_hbm, out_hbm, xv, idx_v, rows_v, sem):
        s = lax.axis_index("s")
        base = s * _GPW
        pltpu.sync_copy(x_hbm.at[pl.ds(base, _GPW)], xv)
        for half in range(_GPW // 16):
            v = xv[pl.ds(16 * half, 16)].astype(jnp.int32)
            idx_v[pl.ds(16 * half, 16)] = jnp.minimum(
                jnp.maximum(v, 0), VOCAB - 1
            )
        pltpu.async_copy(table_hbm.at[idx_v], rows_v, sem).wait()
        pltpu.sync_copy(rows_v, out_hbm.at[pl.ds(base, _GPW), :])

    return _sc_gather


def _dot_t(a, w):
    return lax.dot_general(
        a, w, (((1,), (1,)), ((), ())), preferred_element_type=jnp.float32
    )


def _mlp_body(e_ref, x_ref, w1_ref, b1_ref, w2_ref, b2_ref, w3_ref, b3_ref,
              w4_ref, b4_ref, out_ref):
    e = e_ref[...]
    w1 = w1_ref[...]
    w2, w3, w4 = w2_ref[...], w3_ref[...], w4_ref[...]
    b1, b2, b3 = b1_ref[...], b2_ref[...], b3_ref[...]
    x = x_ref[...]
    q = lax.broadcasted_iota(jnp.int32, (BATCH // 2, BATCH), 0)
    b = lax.broadcasted_iota(jnp.int32, (BATCH // 2, BATCH), 1)
    outs = []
    for p in range(2):
        emb = e[:, 64 * p : 64 * p + 2 * EMB_DIM]
        sel = (b == 2 * q + p).astype(jnp.float32)
        wr = sel @ x[:, 2:4]
        h = _dot_t(emb, w1[:, : 2 * EMB_DIM])
        h = h + _dot_t(wr, w1[:, 2 * EMB_DIM :]) + b1[None, :]
        h = jnp.maximum(h, 0.0)
        h = jnp.maximum(_dot_t(h, w2) + b2[None, :], 0.0)
        h = jnp.maximum(_dot_t(h, w3) + b3[None, :], 0.0)
        outs.append(jnp.sum(h * w4, axis=1, keepdims=True) + b4_ref[0])
    out_ref[...] = jnp.concatenate(outs, axis=1)


def kernel(x, emb_table, W1, b1, W2, b2, W3, b3, W4, b4):
    rows = _sc_gather_fn()(emb_table, x.reshape(4 * BATCH))
    e = rows.reshape(BATCH // 2, 4 * 2 * EMB_DIM)
    out = pl.pallas_call(
        _mlp_body,
        out_shape=jax.ShapeDtypeStruct((BATCH // 2, 2), jnp.float32),
    )(e, x, W1, b1, W2, b2, W3, b3, W4, b4)
    return out.reshape(BATCH, 1)
